# baseline (speedup 1.0000x reference)
import functools

import jax
import jax.numpy as jnp
from jax import lax
from jax.experimental import pallas as pl
from jax.experimental.pallas import tpu as pltpu
from jax.experimental.pallas import tpu_sc as plsc

VOCAB = 100000
EMBED_DIM = 64
BATCH = 1024

_NC = 2
_NS = 16
_NW = _NC * _NS
_B_PER_W = BATCH // _NW


def _sc_gather(emb_weight, context_ids):
    mesh = plsc.VectorSubcoreMesh(
        core_axis_name="c", subcore_axis_name="s",
        num_cores=_NC, num_subcores=_NS,
    )

    @functools.partial(
        pl.kernel,
        out_type=jax.ShapeDtypeStruct((BATCH, EMBED_DIM), jnp.float32),
        mesh=mesh,
        scratch_types=[
            pltpu.VMEM((_B_PER_W,), jnp.int32),
            pltpu.VMEM((_B_PER_W, EMBED_DIM), jnp.float32),
            pltpu.SemaphoreType.DMA,
        ],
        compiler_params=pltpu.CompilerParams(use_tc_tiling_on_sc=False),
    )
    def gather_kernel(table_hbm, idx_hbm, out_hbm, idx_v, rows_v, sem):
        wid = lax.axis_index("s") * _NC + lax.axis_index("c")
        base = wid * _B_PER_W
        pltpu.sync_copy(idx_hbm.at[pl.ds(base, _B_PER_W)], idx_v)
        pltpu.async_copy(table_hbm.at[idx_v], rows_v, sem).wait()
        pltpu.sync_copy(rows_v, out_hbm.at[pl.ds(base, _B_PER_W)])

    return gather_kernel(emb_weight, context_ids)


_BN = 4096


def _tc_matmul_t(embeds, lin_weight):
    nblocks = pl.cdiv(VOCAB, _BN)

    def mm_kernel(lin_ref, emb_ref, out_ref):
        out_ref[...] = jax.lax.dot_general(
            lin_ref[...], emb_ref[...],
            dimension_numbers=(((0,), (1,)), ((), ())),
            preferred_element_type=jnp.float32,
        )

    return pl.pallas_call(
        mm_kernel,
        grid=(nblocks,),
        in_specs=[
            pl.BlockSpec((EMBED_DIM, _BN), lambda j: (0, j)),
            pl.BlockSpec((BATCH, EMBED_DIM), lambda j: (0, 0)),
        ],
        out_specs=pl.BlockSpec((_BN, BATCH), lambda j: (j, 0)),
        out_shape=jax.ShapeDtypeStruct((VOCAB, BATCH), jnp.float32),
        compiler_params=pltpu.CompilerParams(
            dimension_semantics=("arbitrary",),
        ),
    )(lin_weight, embeds)


def kernel(context_ids, emb_weight, lin_weight):
    ids = context_ids.astype(jnp.int32)
    emb_lin = jax.lax.optimization_barrier(jnp.ravel(emb_weight))
    emb_2d = emb_lin.reshape(VOCAB, EMBED_DIM)
    embeds = _sc_gather(emb_2d, ids)
    out_t = _tc_matmul_t(embeds, lin_weight)
    return out_t.T


# speedup vs baseline: 1.0468x; 1.0468x over previous
import functools

import jax
import jax.numpy as jnp
from jax import lax
from jax.experimental import pallas as pl
from jax.experimental.pallas import tpu as pltpu
from jax.experimental.pallas import tpu_sc as plsc

VOCAB = 100000
EMBED_DIM = 64
BATCH = 1024

_NC = 2
_NS = 16
_NW = _NC * _NS
_B_PER_W = BATCH // _NW

_PK = 1024


def _tc_pack(emb_t):
    nblocks = pl.cdiv(VOCAB // 2, _PK)

    def body(in_ref, out_ref):
        y = in_ref[...].T
        y3 = y.reshape(_PK, 2, EMBED_DIM)
        out_ref[...] = jnp.concatenate([y3[:, 0, :], y3[:, 1, :]], axis=1)

    return pl.pallas_call(
        body,
        grid=(nblocks,),
        in_specs=[pl.BlockSpec((EMBED_DIM, 2 * _PK), lambda j: (0, j))],
        out_specs=pl.BlockSpec((_PK, 128), lambda j: (j, 0)),
        out_shape=jax.ShapeDtypeStruct((VOCAB // 2, 128), jnp.float32),
        compiler_params=pltpu.CompilerParams(
            dimension_semantics=("arbitrary",),
        ),
    )(emb_t)


def _sc_gather(packed, context_ids):
    mesh = plsc.VectorSubcoreMesh(
        core_axis_name="c", subcore_axis_name="s",
        num_cores=_NC, num_subcores=_NS,
    )

    @functools.partial(
        pl.kernel,
        out_type=jax.ShapeDtypeStruct((BATCH, EMBED_DIM), jnp.float32),
        mesh=mesh,
        scratch_types=[
            pltpu.VMEM((_B_PER_W,), jnp.int32),
            pltpu.VMEM((_B_PER_W,), jnp.int32),
            pltpu.VMEM((_B_PER_W, 128), jnp.float32),
            pltpu.VMEM((_B_PER_W, EMBED_DIM), jnp.float32),
            pltpu.SemaphoreType.DMA,
        ],
    )
    def gather_kernel(table_hbm, idx_hbm, out_hbm, idx_v, row_v, gath_v, sel_v, sem):
        wid = lax.axis_index("s") * _NC + lax.axis_index("c")
        base = wid * _B_PER_W
        pltpu.sync_copy(idx_hbm.at[pl.ds(base, _B_PER_W)], idx_v)
        ids_vecs = []
        for h in range(_B_PER_W // 16):
            ids_vec = idx_v[pl.ds(h * 16, 16)]
            ids_vecs.append(ids_vec)
            row_v[pl.ds(h * 16, 16)] = ids_vec >> 1
        pltpu.async_copy(table_hbm.at[row_v], gath_v, sem).wait()
        for b in range(_B_PER_W):
            off = (ids_vecs[b // 16][b % 16] & 1) * EMBED_DIM
            for c in range(EMBED_DIM // 16):
                sel_v[b, pl.ds(c * 16, 16)] = gath_v[b, pl.ds(off + c * 16, 16)]
        pltpu.sync_copy(sel_v, out_hbm.at[pl.ds(base, _B_PER_W)])

    return gather_kernel(packed, context_ids)


_BN = 4096


def _tc_matmul_t(embeds, lin_weight):
    nblocks = pl.cdiv(VOCAB, _BN)

    def mm_kernel(lin_ref, emb_ref, out_ref):
        out_ref[...] = jax.lax.dot_general(
            lin_ref[...], emb_ref[...],
            dimension_numbers=(((0,), (1,)), ((), ())),
            preferred_element_type=jnp.float32,
        )

    return pl.pallas_call(
        mm_kernel,
        grid=(nblocks,),
        in_specs=[
            pl.BlockSpec((EMBED_DIM, _BN), lambda j: (0, j)),
            pl.BlockSpec((BATCH, EMBED_DIM), lambda j: (0, 0)),
        ],
        out_specs=pl.BlockSpec((_BN, BATCH), lambda j: (j, 0)),
        out_shape=jax.ShapeDtypeStruct((VOCAB, BATCH), jnp.float32),
        compiler_params=pltpu.CompilerParams(
            dimension_semantics=("arbitrary",),
        ),
    )(lin_weight, embeds)


def kernel(context_ids, emb_weight, lin_weight):
    ids = context_ids.astype(jnp.int32)
    packed = _tc_pack(emb_weight.T)
    embeds = _sc_gather(packed, ids)
    out_t = _tc_matmul_t(embeds, lin_weight)
    return out_t.T
